# Initial kernel scaffold; baseline (speedup 1.0000x reference)
import jax
import jax.numpy as jnp
from jax.experimental import pallas as pl


def kernel(xyz, features, W1, b1, gamma1, beta1, W2, b2, gamma2, beta2):
    raise NotImplementedError("write your pallas kernel here")


# speedup vs baseline: 7.1639x; 7.1639x over previous
import functools

import jax
import jax.numpy as jnp
from jax import lax
from jax.experimental import pallas as pl
from jax.experimental.pallas import tpu as pltpu
from jax.experimental.pallas import tpu_sc as plsc

S_OUT = 1024
KNN = 32
BIG = 1e10


def _fps_body(xyz_ref, cx_ref, cy_ref, cz_ref):
    B, N = xyz_ref.shape[1], xyz_ref.shape[2]
    x0 = xyz_ref[0]
    x1 = xyz_ref[1]
    x2 = xyz_ref[2]
    iota_n = lax.broadcasted_iota(jnp.int32, (B, N), 1)
    iota_s = lax.broadcasted_iota(jnp.int32, (B, S_OUT), 1)

    def step(i, carry):
        dist, far, cx, cy, cz = carry
        onehot = iota_n == far
        c0 = jnp.sum(jnp.where(onehot, x0, 0.0), axis=1, keepdims=True)
        c1 = jnp.sum(jnp.where(onehot, x1, 0.0), axis=1, keepdims=True)
        c2 = jnp.sum(jnp.where(onehot, x2, 0.0), axis=1, keepdims=True)
        mask = (iota_s == i).astype(jnp.float32)
        cx = cx + mask * c0
        cy = cy + mask * c1
        cz = cz + mask * c2
        d0 = x0 - c0
        d1 = x1 - c1
        d2 = x2 - c2
        d = d0 * d0 + d1 * d1 + d2 * d2
        dist = jnp.minimum(dist, d)
        m = jnp.max(dist, axis=1, keepdims=True)
        far = jnp.min(jnp.where(dist == m, iota_n, N), axis=1, keepdims=True)
        return dist, far, cx, cy, cz

    dist0 = jnp.full((B, N), BIG, dtype=jnp.float32)
    far0 = jnp.zeros((B, 1), dtype=jnp.int32)
    cf0 = jnp.zeros((B, S_OUT), dtype=jnp.float32)
    _, _, cx, cy, cz = lax.fori_loop(
        0, S_OUT, step, (dist0, far0, cf0, cf0, cf0))
    cx_ref[...] = cx
    cy_ref[...] = cy
    cz_ref[...] = cz


def _fps(xyz_r):
    B, N = xyz_r.shape[1], xyz_r.shape[2]
    return pl.pallas_call(
        _fps_body,
        out_shape=(
            jax.ShapeDtypeStruct((B, S_OUT), jnp.float32),
            jax.ShapeDtypeStruct((B, S_OUT), jnp.float32),
            jax.ShapeDtypeStruct((B, S_OUT), jnp.float32),
        ),
    )(xyz_r)


_TS = 256


def _knn_body(nx_ref, xyz_ref, idx_ref):
    N = xyz_ref.shape[1]
    b = pl.program_id(0)
    xb = xyz_ref[0]
    sb = nx_ref[0]
    iota4 = lax.broadcasted_iota(jnp.int32, (N, 4), 1)
    x2 = jnp.sum(xb * xb, axis=1, keepdims=True)
    xb_aug = jnp.where(iota4 == 3, x2, xb)
    iota4s = lax.broadcasted_iota(jnp.int32, (_TS, 4), 1)
    sb_aug = jnp.where(iota4s == 3, 1.0, -2.0 * sb)
    s2 = jnp.sum(sb * sb, axis=1, keepdims=True)
    d = lax.dot_general(sb_aug, xb_aug, (((1,), (1,)), ((), ())),
                        precision=lax.Precision.HIGHEST,
                        preferred_element_type=jnp.float32) + s2
    iota_n = lax.broadcasted_iota(jnp.int32, (_TS, N), 1)
    iota_k = lax.broadcasted_iota(jnp.int32, (_TS, KNN), 1)

    def step(k, carry):
        d, sel = carry
        m = jnp.min(d, axis=1, keepdims=True)
        j = jnp.min(jnp.where(d == m, iota_n, N), axis=1, keepdims=True)
        sel = sel + (iota_k == k).astype(jnp.int32) * j
        d = jnp.where(iota_n == j, BIG, d)
        return d, sel

    sel0 = jnp.zeros((_TS, KNN), dtype=jnp.int32)
    _, sel = lax.fori_loop(0, KNN, step, (d, sel0))
    idx_ref[0] = sel + b * N


def _knn(nxp, xyzp):
    B, N = xyzp.shape[0], xyzp.shape[1]
    grid = (B, S_OUT // _TS)
    return pl.pallas_call(
        _knn_body,
        grid=grid,
        in_specs=[
            pl.BlockSpec((1, _TS, 4), lambda b, s: (b, s, 0)),
            pl.BlockSpec((1, N, 4), lambda b, s: (b, 0, 0)),
        ],
        out_specs=pl.BlockSpec((1, _TS, KNN), lambda b, s: (b, s, 0)),
        out_shape=jax.ShapeDtypeStruct((B, S_OUT, KNN), jnp.int32),
    )(nxp, xyzp)


_CH = 128


def _sc_gather(table, idx_flat, nxp16):
    M, D = idx_flat.shape[0], table.shape[1]
    info = plsc.get_sparse_core_info()
    nc, ns = info.num_cores, info.num_subcores
    nw = nc * ns
    rows_w = M // nw
    n_chunks = rows_w // _CH
    mesh = plsc.VectorSubcoreMesh(core_axis_name="c", subcore_axis_name="s")

    @functools.partial(
        pl.kernel,
        mesh=mesh,
        out_type=jax.ShapeDtypeStruct((M, D), jnp.float32),
        scratch_types=[
            pltpu.VMEM((_CH,), jnp.int32),
            pltpu.VMEM((_CH, D), jnp.float32),
            pltpu.VMEM((_CH // KNN, 16), jnp.float32),
            pltpu.SemaphoreType.DMA,
        ],
        compiler_params=pltpu.CompilerParams(use_tc_tiling_on_sc=False),
    )
    def k(table_hbm, idx_hbm, nx_hbm, out_hbm, idx_v, rows_v, nx_v, sem):
        wid = lax.axis_index("s") * nc + lax.axis_index("c")
        base = wid * rows_w

        def chunk(ci, _):
            start = pl.multiple_of(base + ci * _CH, _CH)
            pltpu.sync_copy(idx_hbm.at[pl.ds(start, _CH)], idx_v)
            pltpu.async_copy(table_hbm.at[idx_v], rows_v, sem).wait()
            g0 = pl.multiple_of(start // KNN, _CH // KNN)
            pltpu.sync_copy(nx_hbm.at[pl.ds(g0, _CH // KNN)], nx_v)
            for g in range(_CH // KNN):
                v = nx_v[g, :]
                for kk in range(KNN):
                    r = g * KNN + kk
                    rows_v[r, 0:16] = rows_v[r, 0:16] - v
            pltpu.sync_copy(rows_v, out_hbm.at[pl.ds(start, _CH)])
            return 0

        lax.fori_loop(0, n_chunks, chunk, 0)

    return k(table, idx_flat, nxp16)


_RT1 = 4096


def _stats1_body(x_ref, w_ref, gb_ref, a1_ref, c1_ref, s_ref, ss_ref):
    i = pl.program_id(0)
    n = pl.num_programs(0)

    @pl.when(i == 0)
    def _():
        s_ref[...] = jnp.zeros_like(s_ref)
        ss_ref[...] = jnp.zeros_like(ss_ref)

    y = lax.dot_general(x_ref[...], w_ref[...], (((1,), (1,)), ((), ())),
                        precision=lax.Precision.HIGHEST,
                        preferred_element_type=jnp.float32)
    s_ref[...] += jnp.sum(y, axis=0, keepdims=True)
    ss_ref[...] += jnp.sum(y * y, axis=0, keepdims=True)

    @pl.when(i == n - 1)
    def _():
        m = jnp.float32(n * _RT1)
        mu = s_ref[...] / m
        var = ss_ref[...] / m - mu * mu
        rstd = lax.rsqrt(var + 1e-5)
        a1 = gb_ref[0:1, :] * rstd
        a1_ref[...] = a1
        c1_ref[...] = gb_ref[1:2, :] - mu * a1


def _stats1(xg, w1p, g1b1):
    M = xg.shape[0]
    C1 = w1p.shape[0]
    grid = (M // _RT1,)
    return pl.pallas_call(
        _stats1_body,
        grid=grid,
        in_specs=[
            pl.BlockSpec((_RT1, xg.shape[1]), lambda i: (i, 0)),
            pl.BlockSpec(w1p.shape, lambda i: (0, 0)),
            pl.BlockSpec(g1b1.shape, lambda i: (0, 0)),
        ],
        out_specs=(
            pl.BlockSpec((1, C1), lambda i: (0, 0)),
            pl.BlockSpec((1, C1), lambda i: (0, 0)),
        ),
        out_shape=(
            jax.ShapeDtypeStruct((1, C1), jnp.float32),
            jax.ShapeDtypeStruct((1, C1), jnp.float32),
        ),
        scratch_shapes=[
            pltpu.VMEM((1, C1), jnp.float32),
            pltpu.VMEM((1, C1), jnp.float32),
        ],
        compiler_params=pltpu.CompilerParams(
            dimension_semantics=("arbitrary",)),
    )(xg, w1p, g1b1)


_RT2 = 2048


def _main_body(x_ref, w1_ref, a1_ref, c1_ref, w2_ref, gb2_ref,
               mx_ref, mn_ref, a2_ref, c2_ref, s_ref, ss_ref):
    i = pl.program_id(0)
    n = pl.num_programs(0)

    @pl.when(i == 0)
    def _():
        s_ref[...] = jnp.zeros_like(s_ref)
        ss_ref[...] = jnp.zeros_like(ss_ref)

    y1 = lax.dot_general(x_ref[...], w1_ref[...], (((1,), (1,)), ((), ())),
                         precision=lax.Precision.HIGHEST,
                         preferred_element_type=jnp.float32)
    z1 = jnp.maximum(y1 * a1_ref[...] + c1_ref[...], 0.0)
    y2 = lax.dot_general(z1, w2_ref[...], (((1,), (1,)), ((), ())),
                         precision=lax.Precision.HIGHEST,
                         preferred_element_type=jnp.float32)
    s_ref[...] += jnp.sum(y2, axis=0, keepdims=True)
    ss_ref[...] += jnp.sum(y2 * y2, axis=0, keepdims=True)
    C2 = y2.shape[1]
    y3 = y2.reshape(_RT2 // KNN, KNN, C2)
    mx_ref[...] = jnp.max(y3, axis=1)
    mn_ref[...] = jnp.min(y3, axis=1)

    @pl.when(i == n - 1)
    def _():
        m = jnp.float32(n * _RT2)
        mu = s_ref[...] / m
        var = ss_ref[...] / m - mu * mu
        rstd = lax.rsqrt(var + 1e-5)
        a2 = gb2_ref[0:1, :] * rstd
        a2_ref[...] = a2
        c2_ref[...] = gb2_ref[1:2, :] - mu * a2


def _main(xg, w1p, a1, c1, w2, g2b2):
    M, C0 = xg.shape
    C1 = w1p.shape[0]
    C2 = w2.shape[0]
    grid = (M // _RT2,)
    G = _RT2 // KNN
    return pl.pallas_call(
        _main_body,
        grid=grid,
        in_specs=[
            pl.BlockSpec((_RT2, C0), lambda i: (i, 0)),
            pl.BlockSpec((C1, C0), lambda i: (0, 0)),
            pl.BlockSpec((1, C1), lambda i: (0, 0)),
            pl.BlockSpec((1, C1), lambda i: (0, 0)),
            pl.BlockSpec((C2, C1), lambda i: (0, 0)),
            pl.BlockSpec((2, C2), lambda i: (0, 0)),
        ],
        out_specs=(
            pl.BlockSpec((G, C2), lambda i: (i, 0)),
            pl.BlockSpec((G, C2), lambda i: (i, 0)),
            pl.BlockSpec((1, C2), lambda i: (0, 0)),
            pl.BlockSpec((1, C2), lambda i: (0, 0)),
        ),
        out_shape=(
            jax.ShapeDtypeStruct((M // KNN, C2), jnp.float32),
            jax.ShapeDtypeStruct((M // KNN, C2), jnp.float32),
            jax.ShapeDtypeStruct((1, C2), jnp.float32),
            jax.ShapeDtypeStruct((1, C2), jnp.float32),
        ),
        scratch_shapes=[
            pltpu.VMEM((1, C2), jnp.float32),
            pltpu.VMEM((1, C2), jnp.float32),
        ],
        compiler_params=pltpu.CompilerParams(
            dimension_semantics=("arbitrary",)),
    )(xg, w1p, a1, c1, w2, g2b2)


_RT3 = 1024


def _finish_body(mx_ref, mn_ref, a2_ref, c2_ref, o_ref):
    a2 = a2_ref[...]
    picked = jnp.where(a2 >= 0.0, mx_ref[...], mn_ref[...])
    o_ref[...] = jnp.maximum(picked * a2 + c2_ref[...], 0.0)


def _finish(mx, mn, a2, c2):
    G, C2 = mx.shape
    grid = (G // _RT3,)
    return pl.pallas_call(
        _finish_body,
        grid=grid,
        in_specs=[
            pl.BlockSpec((_RT3, C2), lambda i: (i, 0)),
            pl.BlockSpec((_RT3, C2), lambda i: (i, 0)),
            pl.BlockSpec((1, C2), lambda i: (0, 0)),
            pl.BlockSpec((1, C2), lambda i: (0, 0)),
        ],
        out_specs=pl.BlockSpec((_RT3, C2), lambda i: (i, 0)),
        out_shape=jax.ShapeDtypeStruct((G, C2), jnp.float32),
    )(mx, mn, a2, c2)


def kernel(xyz, features, W1, b1, gamma1, beta1, W2, b2, gamma2, beta2):
    B, N, _ = xyz.shape
    D = features.shape[2]
    C1, C0 = W1.shape
    C2 = W2.shape[0]
    C0P = 144

    xyz_r = jnp.transpose(xyz, (2, 0, 1))
    cx, cy, cz = _fps(xyz_r)
    new_xyz = jnp.stack([cx, cy, cz], axis=-1)

    zeros_b1 = jnp.zeros((B, S_OUT, 1), jnp.float32)
    nxp = jnp.concatenate([new_xyz, zeros_b1], axis=-1)
    xyzp = jnp.concatenate([xyz, jnp.zeros((B, N, 1), jnp.float32)], axis=-1)
    idx = _knn(nxp, xyzp)
    idx_flat = idx.reshape(-1)

    table = jnp.concatenate(
        [xyz, features, jnp.zeros((B, N, C0P - 3 - D), jnp.float32)],
        axis=-1).reshape(B * N, C0P)
    nxp16 = jnp.concatenate(
        [new_xyz.reshape(B * S_OUT, 3),
         jnp.zeros((B * S_OUT, 13), jnp.float32)], axis=-1)
    xg = _sc_gather(table, idx_flat, nxp16)

    w1p = jnp.concatenate([W1, jnp.zeros((C1, C0P - C0), jnp.float32)], axis=1)
    g1b1 = jnp.stack([gamma1, beta1])
    g2b2 = jnp.stack([gamma2, beta2])
    a1, c1 = _stats1(xg, w1p, g1b1)
    mx, mn, a2, c2 = _main(xg, w1p, a1, c1, W2, g2b2)
    xout = _finish(mx, mn, a2, c2).reshape(B, S_OUT, C2)
    return (new_xyz, xout)


# speedup vs baseline: 8.5326x; 1.1911x over previous
import functools

import jax
import jax.numpy as jnp
from jax import lax
from jax.experimental import pallas as pl
from jax.experimental.pallas import tpu as pltpu
from jax.experimental.pallas import tpu_sc as plsc

S_OUT = 1024
KNN = 32
BIG = 1e10


def _fps_body(xyz_ref, cx_ref, cy_ref, cz_ref):
    B, N = xyz_ref.shape[1], xyz_ref.shape[2]
    x0 = xyz_ref[0]
    x1 = xyz_ref[1]
    x2 = xyz_ref[2]
    iota_n = lax.broadcasted_iota(jnp.int32, (B, N), 1)
    iota_s = lax.broadcasted_iota(jnp.int32, (B, S_OUT), 1)

    def step(i, carry):
        dist, far, cx, cy, cz = carry
        onehot = iota_n == far
        c0 = jnp.sum(jnp.where(onehot, x0, 0.0), axis=1, keepdims=True)
        c1 = jnp.sum(jnp.where(onehot, x1, 0.0), axis=1, keepdims=True)
        c2 = jnp.sum(jnp.where(onehot, x2, 0.0), axis=1, keepdims=True)
        mask = (iota_s == i).astype(jnp.float32)
        cx = cx + mask * c0
        cy = cy + mask * c1
        cz = cz + mask * c2
        d0 = x0 - c0
        d1 = x1 - c1
        d2 = x2 - c2
        d = d0 * d0 + d1 * d1 + d2 * d2
        dist = jnp.minimum(dist, d)
        m = jnp.max(dist, axis=1, keepdims=True)
        far = jnp.min(jnp.where(dist == m, iota_n, N), axis=1, keepdims=True)
        return dist, far, cx, cy, cz

    dist0 = jnp.full((B, N), BIG, dtype=jnp.float32)
    far0 = jnp.zeros((B, 1), dtype=jnp.int32)
    cf0 = jnp.zeros((B, S_OUT), dtype=jnp.float32)
    _, _, cx, cy, cz = lax.fori_loop(
        0, S_OUT, step, (dist0, far0, cf0, cf0, cf0))
    cx_ref[...] = cx
    cy_ref[...] = cy
    cz_ref[...] = cz


def _fps(xyz_r):
    B, N = xyz_r.shape[1], xyz_r.shape[2]
    return pl.pallas_call(
        _fps_body,
        out_shape=(
            jax.ShapeDtypeStruct((B, S_OUT), jnp.float32),
            jax.ShapeDtypeStruct((B, S_OUT), jnp.float32),
            jax.ShapeDtypeStruct((B, S_OUT), jnp.float32),
        ),
    )(xyz_r)


_TS = 256


def _knn_body(nx_ref, xyz_ref, idx_ref):
    N = xyz_ref.shape[1]
    b = pl.program_id(0)
    xb = xyz_ref[0]
    sb = nx_ref[0]
    iota4 = lax.broadcasted_iota(jnp.int32, (N, 4), 1)
    x2 = jnp.sum(xb * xb, axis=1, keepdims=True)
    xb_aug = jnp.where(iota4 == 3, x2, xb)
    iota4s = lax.broadcasted_iota(jnp.int32, (_TS, 4), 1)
    sb_aug = jnp.where(iota4s == 3, 1.0, -2.0 * sb)
    s2 = jnp.sum(sb * sb, axis=1, keepdims=True)
    d = lax.dot_general(sb_aug, xb_aug, (((1,), (1,)), ((), ())),
                        precision=lax.Precision.HIGHEST,
                        preferred_element_type=jnp.float32) + s2
    iota_n = lax.broadcasted_iota(jnp.int32, (_TS, N), 1)
    iota_k = lax.broadcasted_iota(jnp.int32, (_TS, KNN), 1)

    def step(k, carry):
        d, sel = carry
        m = jnp.min(d, axis=1, keepdims=True)
        j = jnp.min(jnp.where(d == m, iota_n, N), axis=1, keepdims=True)
        sel = sel + (iota_k == k).astype(jnp.int32) * j
        d = jnp.where(iota_n == j, BIG, d)
        return d, sel

    sel0 = jnp.zeros((_TS, KNN), dtype=jnp.int32)
    _, sel = lax.fori_loop(0, KNN, step, (d, sel0))
    idx_ref[0] = sel + b * N


def _knn(nxp, xyzp):
    B, N = xyzp.shape[0], xyzp.shape[1]
    grid = (B, S_OUT // _TS)
    return pl.pallas_call(
        _knn_body,
        grid=grid,
        in_specs=[
            pl.BlockSpec((1, _TS, 4), lambda b, s: (b, s, 0)),
            pl.BlockSpec((1, N, 4), lambda b, s: (b, 0, 0)),
        ],
        out_specs=pl.BlockSpec((1, _TS, KNN), lambda b, s: (b, s, 0)),
        out_shape=jax.ShapeDtypeStruct((B, S_OUT, KNN), jnp.int32),
    )(nxp, xyzp)


_CH = 128


def _sc_gather(table, idx_flat, nxp16):
    M, D = idx_flat.shape[0], table.shape[1]
    info = plsc.get_sparse_core_info()
    nc, ns = info.num_cores, info.num_subcores
    nw = nc * ns
    rows_w = M // nw
    n_chunks = rows_w // _CH
    mesh = plsc.VectorSubcoreMesh(core_axis_name="c", subcore_axis_name="s")

    @functools.partial(
        pl.kernel,
        mesh=mesh,
        out_type=jax.ShapeDtypeStruct((M, D), jnp.float32),
        scratch_types=[
            pltpu.VMEM((_CH,), jnp.int32),
            pltpu.VMEM((_CH, D), jnp.float32),
            pltpu.VMEM((_CH // KNN, 16), jnp.float32),
            pltpu.SemaphoreType.DMA,
        ],
        compiler_params=pltpu.CompilerParams(use_tc_tiling_on_sc=False),
    )
    def k(table_hbm, idx_hbm, nx_hbm, out_hbm, idx_v, rows_v, nx_v, sem):
        wid = lax.axis_index("s") * nc + lax.axis_index("c")
        base = wid * rows_w

        def chunk(ci, _):
            start = pl.multiple_of(base + ci * _CH, _CH)
            pltpu.sync_copy(idx_hbm.at[pl.ds(start, _CH)], idx_v)
            pltpu.async_copy(table_hbm.at[idx_v], rows_v, sem).wait()
            g0 = pl.multiple_of(start // KNN, _CH // KNN)
            pltpu.sync_copy(nx_hbm.at[pl.ds(g0, _CH // KNN)], nx_v)
            for g in range(_CH // KNN):
                v = nx_v[g, :]
                for kk in range(KNN):
                    r = g * KNN + kk
                    rows_v[r, 0:16] = rows_v[r, 0:16] - v
            pltpu.sync_copy(rows_v, out_hbm.at[pl.ds(start, _CH)])
            return 0

        lax.fori_loop(0, n_chunks, chunk, 0)

    return k(table, idx_flat, nxp16)


_RT1 = 4096


def _stats1_body(x_ref, w_ref, gb_ref, a1_ref, c1_ref, s_ref, ss_ref):
    i = pl.program_id(0)
    n = pl.num_programs(0)

    @pl.when(i == 0)
    def _():
        s_ref[...] = jnp.zeros_like(s_ref)
        ss_ref[...] = jnp.zeros_like(ss_ref)

    y = lax.dot_general(x_ref[...], w_ref[...], (((1,), (1,)), ((), ())),
                        preferred_element_type=jnp.float32)
    s_ref[...] += jnp.sum(y, axis=0, keepdims=True)
    ss_ref[...] += jnp.sum(y * y, axis=0, keepdims=True)

    @pl.when(i == n - 1)
    def _():
        m = jnp.float32(n * _RT1)
        mu = s_ref[...] / m
        var = ss_ref[...] / m - mu * mu
        rstd = lax.rsqrt(var + 1e-5)
        a1 = gb_ref[0:1, :] * rstd
        a1_ref[...] = a1
        c1_ref[...] = gb_ref[1:2, :] - mu * a1


def _stats1(xg, w1p, g1b1):
    M = xg.shape[0]
    C1 = w1p.shape[0]
    grid = (M // _RT1,)
    return pl.pallas_call(
        _stats1_body,
        grid=grid,
        in_specs=[
            pl.BlockSpec((_RT1, xg.shape[1]), lambda i: (i, 0)),
            pl.BlockSpec(w1p.shape, lambda i: (0, 0)),
            pl.BlockSpec(g1b1.shape, lambda i: (0, 0)),
        ],
        out_specs=(
            pl.BlockSpec((1, C1), lambda i: (0, 0)),
            pl.BlockSpec((1, C1), lambda i: (0, 0)),
        ),
        out_shape=(
            jax.ShapeDtypeStruct((1, C1), jnp.float32),
            jax.ShapeDtypeStruct((1, C1), jnp.float32),
        ),
        scratch_shapes=[
            pltpu.VMEM((1, C1), jnp.float32),
            pltpu.VMEM((1, C1), jnp.float32),
        ],
        compiler_params=pltpu.CompilerParams(
            dimension_semantics=("arbitrary",)),
    )(xg, w1p, g1b1)


_RT2 = 2048


def _main_body(x_ref, w1_ref, a1_ref, c1_ref, w2_ref, gb2_ref,
               mx_ref, mn_ref, a2_ref, c2_ref, s_ref, ss_ref):
    i = pl.program_id(0)
    n = pl.num_programs(0)

    @pl.when(i == 0)
    def _():
        s_ref[...] = jnp.zeros_like(s_ref)
        ss_ref[...] = jnp.zeros_like(ss_ref)

    y1 = lax.dot_general(x_ref[...], w1_ref[...], (((1,), (1,)), ((), ())),
                         preferred_element_type=jnp.float32)
    z1 = jnp.maximum(y1 * a1_ref[...] + c1_ref[...], 0.0)
    y2 = lax.dot_general(z1, w2_ref[...], (((1,), (1,)), ((), ())),
                         preferred_element_type=jnp.float32)
    s_ref[...] += jnp.sum(y2, axis=0, keepdims=True)
    ss_ref[...] += jnp.sum(y2 * y2, axis=0, keepdims=True)
    C2 = y2.shape[1]
    y3 = y2.reshape(_RT2 // KNN, KNN, C2)
    mx_ref[...] = jnp.max(y3, axis=1)
    mn_ref[...] = jnp.min(y3, axis=1)

    @pl.when(i == n - 1)
    def _():
        m = jnp.float32(n * _RT2)
        mu = s_ref[...] / m
        var = ss_ref[...] / m - mu * mu
        rstd = lax.rsqrt(var + 1e-5)
        a2 = gb2_ref[0:1, :] * rstd
        a2_ref[...] = a2
        c2_ref[...] = gb2_ref[1:2, :] - mu * a2


def _main(xg, w1p, a1, c1, w2, g2b2):
    M, C0 = xg.shape
    C1 = w1p.shape[0]
    C2 = w2.shape[0]
    grid = (M // _RT2,)
    G = _RT2 // KNN
    return pl.pallas_call(
        _main_body,
        grid=grid,
        in_specs=[
            pl.BlockSpec((_RT2, C0), lambda i: (i, 0)),
            pl.BlockSpec((C1, C0), lambda i: (0, 0)),
            pl.BlockSpec((1, C1), lambda i: (0, 0)),
            pl.BlockSpec((1, C1), lambda i: (0, 0)),
            pl.BlockSpec((C2, C1), lambda i: (0, 0)),
            pl.BlockSpec((2, C2), lambda i: (0, 0)),
        ],
        out_specs=(
            pl.BlockSpec((G, C2), lambda i: (i, 0)),
            pl.BlockSpec((G, C2), lambda i: (i, 0)),
            pl.BlockSpec((1, C2), lambda i: (0, 0)),
            pl.BlockSpec((1, C2), lambda i: (0, 0)),
        ),
        out_shape=(
            jax.ShapeDtypeStruct((M // KNN, C2), jnp.float32),
            jax.ShapeDtypeStruct((M // KNN, C2), jnp.float32),
            jax.ShapeDtypeStruct((1, C2), jnp.float32),
            jax.ShapeDtypeStruct((1, C2), jnp.float32),
        ),
        scratch_shapes=[
            pltpu.VMEM((1, C2), jnp.float32),
            pltpu.VMEM((1, C2), jnp.float32),
        ],
        compiler_params=pltpu.CompilerParams(
            dimension_semantics=("arbitrary",)),
    )(xg, w1p, a1, c1, w2, g2b2)


_RT3 = 1024


def _finish_body(mx_ref, mn_ref, a2_ref, c2_ref, o_ref):
    a2 = a2_ref[...]
    picked = jnp.where(a2 >= 0.0, mx_ref[...], mn_ref[...])
    o_ref[...] = jnp.maximum(picked * a2 + c2_ref[...], 0.0)


def _finish(mx, mn, a2, c2):
    G, C2 = mx.shape
    grid = (G // _RT3,)
    return pl.pallas_call(
        _finish_body,
        grid=grid,
        in_specs=[
            pl.BlockSpec((_RT3, C2), lambda i: (i, 0)),
            pl.BlockSpec((_RT3, C2), lambda i: (i, 0)),
            pl.BlockSpec((1, C2), lambda i: (0, 0)),
            pl.BlockSpec((1, C2), lambda i: (0, 0)),
        ],
        out_specs=pl.BlockSpec((_RT3, C2), lambda i: (i, 0)),
        out_shape=jax.ShapeDtypeStruct((G, C2), jnp.float32),
    )(mx, mn, a2, c2)


def kernel(xyz, features, W1, b1, gamma1, beta1, W2, b2, gamma2, beta2):
    B, N, _ = xyz.shape
    D = features.shape[2]
    C1, C0 = W1.shape
    C2 = W2.shape[0]
    C0P = 144

    xyz_r = jnp.transpose(xyz, (2, 0, 1))
    cx, cy, cz = _fps(xyz_r)
    new_xyz = jnp.stack([cx, cy, cz], axis=-1)

    zeros_b1 = jnp.zeros((B, S_OUT, 1), jnp.float32)
    nxp = jnp.concatenate([new_xyz, zeros_b1], axis=-1)
    xyzp = jnp.concatenate([xyz, jnp.zeros((B, N, 1), jnp.float32)], axis=-1)
    idx = _knn(nxp, xyzp)
    idx_flat = idx.reshape(-1)

    table = jnp.concatenate(
        [xyz, features, jnp.zeros((B, N, C0P - 3 - D), jnp.float32)],
        axis=-1).reshape(B * N, C0P)
    nxp16 = jnp.concatenate(
        [new_xyz.reshape(B * S_OUT, 3),
         jnp.zeros((B * S_OUT, 13), jnp.float32)], axis=-1)
    xg = _sc_gather(table, idx_flat, nxp16)

    w1p = jnp.concatenate([W1, jnp.zeros((C1, C0P - C0), jnp.float32)], axis=1)
    g1b1 = jnp.stack([gamma1, beta1])
    g2b2 = jnp.stack([gamma2, beta2])
    a1, c1 = _stats1(xg, w1p, g1b1)
    mx, mn, a2, c2 = _main(xg, w1p, a1, c1, W2, g2b2)
    xout = _finish(mx, mn, a2, c2).reshape(B, S_OUT, C2)
    return (new_xyz, xout)
